# baseline (speedup 1.0000x reference)
import functools

import jax
import jax.numpy as jnp
from jax import lax
from jax.experimental import pallas as pl
from jax.experimental.pallas import tpu as pltpu
from jax.experimental.pallas import tpu_sc as plsc

N = 10000
M = 2500
E = 320000
D_IN = 128
D_HID = 128
N_CLS = 40

NC = 2
NS = 16
L = 16

N_PAD = 10240
M_PAD = 4096
NT = N_PAD // NS
MT = M_PAD // NS

EW = E // (NC * NS)
ROWS = EW // L

C = 100
EC = E // NS
NCH = EC // C
NB = 5

BN = 400

_sc_mesh = functools.partial(
    plsc.VectorSubcoreMesh, core_axis_name="c", subcore_axis_name="s",
    num_cores=NC, num_subcores=NS)


def _degrees_body(vidx_hbm, eidx_hbm, dv_out, de_out,
                  idxv, dvl, del_, dvsl, desl, rbv, rbe, rov, roe):
    c = lax.axis_index("c")
    s = lax.axis_index("s")
    w = c * NS + s
    zeros = jnp.zeros((L,), jnp.float32)
    ones = jnp.ones((L,), jnp.float32)

    def zero1d(ref, n):
        def zb(i, _):
            ref[pl.ds(i * L, L)] = zeros
            return 0
        lax.fori_loop(0, n // L, zb, 0)

    zero1d(dvl, N_PAD)
    zero1d(del_, M_PAD)

    pltpu.sync_copy(vidx_hbm.at[w], idxv)

    def hv(i, _):
        plsc.addupdate_scatter(dvl, [idxv[i]], ones)
        return 0
    lax.fori_loop(0, ROWS, hv, 0)

    pltpu.sync_copy(eidx_hbm.at[w], idxv)

    def he(i, _):
        plsc.addupdate_scatter(del_, [idxv[i]], ones)
        return 0
    lax.fori_loop(0, ROWS, he, 0)

    pltpu.sync_copy(dvl, dvsl.at[s])
    pltpu.sync_copy(del_, desl.at[s])
    plsc.subcore_barrier()

    pltpu.sync_copy(dvsl.at[:, pl.ds(s * NT, NT)], rbv)
    pltpu.sync_copy(desl.at[:, pl.ds(s * MT, MT)], rbe)

    def reduce_cols(rb, ro, nt):
        def red(j, _):
            def addk(k, acc):
                return acc + rb[k, pl.ds(j * L, L)]
            ro[pl.ds(j * L, L)] = lax.fori_loop(
                0, NS, addk, jnp.zeros((L,), jnp.float32))
            return 0
        lax.fori_loop(0, nt // L, red, 0)

    reduce_cols(rbv, rov, NT)
    reduce_cols(rbe, roe, MT)
    pltpu.sync_copy(rov, dv_out.at[c, pl.ds(s * NT, NT)])
    pltpu.sync_copy(roe, de_out.at[c, pl.ds(s * MT, MT)])


def _degrees(vidx, eidx):
    vidx_r = vidx.reshape(NC * NS, ROWS, L)
    eidx_r = eidx.reshape(NC * NS, ROWS, L)
    return pl.kernel(
        _degrees_body,
        out_type=[jax.ShapeDtypeStruct((NC, N_PAD), jnp.float32),
                  jax.ShapeDtypeStruct((NC, M_PAD), jnp.float32)],
        mesh=_sc_mesh(),
        compiler_params=pltpu.CompilerParams(needs_layout_passes=False),
        scratch_types=[
            pltpu.VMEM((ROWS, L), jnp.int32),
            pltpu.VMEM((N_PAD,), jnp.float32),
            pltpu.VMEM((M_PAD,), jnp.float32),
            pltpu.VMEM_SHARED((NS, N_PAD), jnp.float32),
            pltpu.VMEM_SHARED((NS, M_PAD), jnp.float32),
            pltpu.VMEM((NS, NT), jnp.float32),
            pltpu.VMEM((NS, MT), jnp.float32),
            pltpu.VMEM((NT,), jnp.float32),
            pltpu.VMEM((MT,), jnp.float32),
        ],
    )(vidx_r, eidx_r)


EWS = E // (NC * NS)
NCH = EWS // C
C2 = 40
NCH2 = EWS // C2
D = 128
MY = 2560
MYT = MY // NS
HALF = N_PAD // 2
HT = HALF // NS
TR = 128


def _zero_rowbuf0(rowbuf):
    zeros = jnp.zeros((L,), jnp.float32)

    def zb(i, _):
        rowbuf[0, i // (D // L), pl.ds((i % (D // L)) * L, L)] = zeros
        return 0
    lax.fori_loop(0, C * D // L, zb, 0)


def _pipeline(rowbuf, gsem, ssem, gsrc, gidx, sdst, sidx, nch):
    def issue_g(j, b):
        pltpu.async_copy(gsrc.at[gidx.at[j]], rowbuf.at[b], gsem.at[b])

    def wait_g(j, b):
        pltpu.make_async_copy(
            gsrc.at[gidx.at[j]], rowbuf.at[b], gsem.at[b]).wait()

    def issue_s(j, b):
        pltpu.async_copy(rowbuf.at[b], sdst.at[sidx.at[j]], ssem.at[b],
                         add=True)

    def wait_s(j, b):
        pltpu.make_async_copy(
            rowbuf.at[b], sdst.at[sidx.at[j]], ssem.at[b]).wait()

    for b in range(NB):
        issue_g(b, b)

    def body(i, _):
        j0 = i * NB
        for b in range(NB):
            wait_g(j0 + b, b)
            issue_s(j0 + b, b)
        for b in range(NB):
            wait_s(j0 + b, b)

            @pl.when(j0 + b + NB < nch)
            def _():
                issue_g(j0 + b + NB, b)
        return 0
    lax.fori_loop(0, nch // NB, body, 0)


def _p1_body(xv_hbm, vidx_hbm, eidx_hbm, ye_out,
             vidx_v, eidx_v, rowbuf, ye_sp, gsem, ssem):
    c = lax.axis_index("c")
    s = lax.axis_index("s")
    w = c * NS + s

    pltpu.sync_copy(vidx_hbm.at[w], vidx_v)
    pltpu.sync_copy(eidx_hbm.at[w], eidx_v)

    _zero_rowbuf0(rowbuf)
    base = s * MYT
    for t in range(MYT // C):
        pltpu.sync_copy(rowbuf.at[0], ye_sp.at[pl.ds(base + t * C, C)])
    if MYT % C:
        pltpu.sync_copy(rowbuf.at[0].at[pl.ds(0, MYT % C)],
                        ye_sp.at[pl.ds(base + (MYT // C) * C, MYT % C)])
    plsc.subcore_barrier()

    _pipeline(rowbuf, gsem, ssem, xv_hbm, vidx_v, ye_sp, eidx_v, NCH)
    plsc.subcore_barrier()

    pltpu.sync_copy(ye_sp.at[pl.ds(base, MYT)],
                    ye_out.at[c, pl.ds(base, MYT)])


def _p2_body(ye_hbm, vidx_hbm, eidx_hbm, xo_out,
             vring, ering, rowbuf, ye_sp, xo_sp,
             gsem, ssem, visem, eisem):
    c = lax.axis_index("c")
    s = lax.axis_index("s")
    w = c * NS + s
    zeros = jnp.zeros((L,), jnp.float32)

    def zb(i, _):
        rowbuf[0, i // (D // L), pl.ds((i % (D // L)) * L, L)] = zeros
        return 0
    lax.fori_loop(0, C2 * D // L, zb, 0)
    nrow = N_PAD // NS
    for t in range(nrow // C2):
        pltpu.sync_copy(rowbuf.at[0], xo_sp.at[pl.ds(s * nrow + t * C2, C2)])

    pltpu.sync_copy(ye_hbm.at[pl.ds(s * MYT, MYT)],
                    ye_sp.at[pl.ds(s * MYT, MYT)])
    plsc.subcore_barrier()

    def issue_i(j, b):
        pltpu.async_copy(vidx_hbm.at[w, j], vring.at[b], visem.at[b])
        pltpu.async_copy(eidx_hbm.at[w, j], ering.at[b], eisem.at[b])

    def wait_i(j, b):
        pltpu.make_async_copy(
            vidx_hbm.at[w, j], vring.at[b], visem.at[b]).wait()
        pltpu.make_async_copy(
            eidx_hbm.at[w, j], ering.at[b], eisem.at[b]).wait()

    def issue_g(j, b):
        pltpu.async_copy(ye_sp.at[ering.at[b]], rowbuf.at[b], gsem.at[b])

    def wait_g(j, b):
        pltpu.make_async_copy(
            ye_sp.at[ering.at[b]], rowbuf.at[b], gsem.at[b]).wait()

    def issue_s(j, b):
        pltpu.async_copy(rowbuf.at[b], xo_sp.at[vring.at[b]], ssem.at[b],
                         add=True)

    def wait_s(j, b):
        pltpu.make_async_copy(
            rowbuf.at[b], xo_sp.at[vring.at[b]], ssem.at[b]).wait()

    for b in range(NB):
        issue_i(b, b)

    def body(i, _):
        j0 = i * NB
        for b in range(NB):
            wait_i(j0 + b, b)
            issue_g(j0 + b, b)
        for b in range(NB):
            wait_g(j0 + b, b)
            issue_s(j0 + b, b)
        for b in range(NB):
            wait_s(j0 + b, b)

            @pl.when(j0 + b + NB < NCH2)
            def _():
                issue_i(j0 + b + NB, b)
        return 0
    lax.fori_loop(0, NCH2 // NB, body, 0)
    plsc.subcore_barrier()

    pltpu.sync_copy(xo_sp.at[pl.ds(s * nrow, nrow)],
                    xo_out.at[c, pl.ds(s * nrow, nrow)])


def _smooth_p1(xv, vidx_r, eidx_r):
    return pl.kernel(
        _p1_body,
        out_type=jax.ShapeDtypeStruct((NC, MY, D), jnp.float32),
        mesh=_sc_mesh(),
        scratch_types=[
            pltpu.VMEM((NCH, C), jnp.int32),
            pltpu.VMEM((NCH, C), jnp.int32),
            pltpu.VMEM((NB, C, D), jnp.float32),
            pltpu.VMEM_SHARED((MY, D), jnp.float32),
            pltpu.SemaphoreType.DMA((NB,)),
            pltpu.SemaphoreType.DMA((NB,)),
        ],
    )(xv, vidx_r, eidx_r)


def _smooth_p2(ye_comb, vidx2_r, eidx2_r):
    return pl.kernel(
        _p2_body,
        out_type=jax.ShapeDtypeStruct((NC, N_PAD, D), jnp.float32),
        mesh=_sc_mesh(),
        scratch_types=[
            pltpu.VMEM((NB, C2), jnp.int32),
            pltpu.VMEM((NB, C2), jnp.int32),
            pltpu.VMEM((NB, C2, D), jnp.float32),
            pltpu.VMEM_SHARED((MY, D), jnp.float32),
            pltpu.VMEM_SHARED((N_PAD, D), jnp.float32),
            pltpu.SemaphoreType.DMA((NB,)),
            pltpu.SemaphoreType.DMA((NB,)),
            pltpu.SemaphoreType.DMA((NB,)),
            pltpu.SemaphoreType.DMA((NB,)),
        ],
    )(ye_comb, vidx2_r, eidx2_r)


def _norm_body(dv_ref, de_ref, dvi_ref, dei_ref):
    dv = dv_ref[0:1, :] + dv_ref[1:2, :]
    de = de_ref[0:1, :] + de_ref[1:2, :]
    dvi_ref[...] = jnp.where(
        dv > 0, lax.rsqrt(jnp.maximum(dv, 1e-12)), 0.0)
    dei_ref[...] = jnp.where(de > 0, 1.0 / jnp.maximum(de, 1e-12), 0.0)


def _normalizers(dv_parts, de_parts):
    return pl.pallas_call(
        _norm_body,
        in_specs=[pl.BlockSpec((NC, N_PAD), lambda: (0, 0)),
                  pl.BlockSpec((NC, M_PAD), lambda: (0, 0))],
        out_specs=[pl.BlockSpec((1, N_PAD), lambda: (0, 0)),
                   pl.BlockSpec((1, M_PAD), lambda: (0, 0))],
        out_shape=[jax.ShapeDtypeStruct((1, N_PAD), jnp.float32),
                   jax.ShapeDtypeStruct((1, M_PAD), jnp.float32)],
    )(dv_parts, de_parts)


def _comb_body(p_ref, s_ref, o_ref):
    o_ref[...] = (p_ref[0] + p_ref[1]) * s_ref[...]


BM = 320


def _combine(ye_parts, dei_col):
    return pl.pallas_call(
        _comb_body,
        grid=(MY // BM,),
        in_specs=[
            pl.BlockSpec((NC, BM, D), lambda i: (0, i, 0)),
            pl.BlockSpec((BM, 1), lambda i: (i, 0)),
        ],
        out_specs=pl.BlockSpec((BM, D), lambda i: (i, 0)),
        out_shape=jax.ShapeDtypeStruct((MY, D), jnp.float32),
    )(ye_parts, dei_col)


def _mmA_body(x_ref, w_ref, b_ref, s_ref, o_ref):
    y = jnp.dot(x_ref[...], w_ref[...], preferred_element_type=jnp.float32)
    o_ref[...] = (y + b_ref[...]) * s_ref[...]


def _matmul_a(x, w, b, dvi):
    return pl.pallas_call(
        _mmA_body,
        grid=(N // BN,),
        in_specs=[
            pl.BlockSpec((BN, D_IN), lambda i: (i, 0)),
            pl.BlockSpec((D_IN, D_HID), lambda i: (0, 0)),
            pl.BlockSpec((1, D_HID), lambda i: (0, 0)),
            pl.BlockSpec((BN, 1), lambda i: (i, 0)),
        ],
        out_specs=pl.BlockSpec((BN, D_HID), lambda i: (i, 0)),
        out_shape=jax.ShapeDtypeStruct((N_PAD, D_HID), jnp.float32),
    )(x, w, b.reshape(1, D_HID), dvi)


def _mmB_body(x_ref, w_ref, b_ref, s_ref, o_ref):
    x = x_ref[0] + x_ref[1]
    h = jax.nn.relu(x * s_ref[...])
    y = jnp.dot(h, w_ref[...], preferred_element_type=jnp.float32)
    y = (y + b_ref[...]) * s_ref[...]
    o_ref[...] = jnp.concatenate(
        [y, jnp.zeros((BN, D_HID - N_CLS), jnp.float32)], axis=1)


def _matmul_b(xo0, w, b, dvi):
    return pl.pallas_call(
        _mmB_body,
        grid=(N // BN,),
        in_specs=[
            pl.BlockSpec((NC, BN, D_HID), lambda i: (0, i, 0)),
            pl.BlockSpec((D_HID, N_CLS), lambda i: (0, 0)),
            pl.BlockSpec((1, N_CLS), lambda i: (0, 0)),
            pl.BlockSpec((BN, 1), lambda i: (i, 0)),
        ],
        out_specs=pl.BlockSpec((BN, D_HID), lambda i: (i, 0)),
        out_shape=jax.ShapeDtypeStruct((N_PAD, D_HID), jnp.float32),
    )(xo0, w, b.reshape(1, N_CLS), dvi)


def _final_body(x_ref, s_ref, o_ref):
    o_ref[...] = (x_ref[0] + x_ref[1]) * s_ref[...]


def _final_scale(xo1, dvi):
    return pl.pallas_call(
        _final_body,
        grid=(N // BN,),
        in_specs=[
            pl.BlockSpec((NC, BN, D_HID), lambda i: (0, i, 0)),
            pl.BlockSpec((BN, 1), lambda i: (i, 0)),
        ],
        out_specs=pl.BlockSpec((BN, D_HID), lambda i: (i, 0)),
        out_shape=jax.ShapeDtypeStruct((N, D_HID), jnp.float32),
    )(xo1, dvi)


def kernel(X, vertex_idx, hyperedge_idx, W0, b0, W1, b1):
    vidx = vertex_idx.astype(jnp.int32)
    eidx = hyperedge_idx.astype(jnp.int32)
    vidx_r = vidx.reshape(NC * NS, NCH, C)
    eidx_r = eidx.reshape(NC * NS, NCH, C)
    vidx2_r = vidx.reshape(NC * NS, NCH2, C2)
    eidx2_r = eidx.reshape(NC * NS, NCH2, C2)

    dv_parts, de_parts = _degrees(vidx, eidx)
    dvi2, dei2 = _normalizers(dv_parts, de_parts)
    dvi = dvi2.reshape(N_PAD, 1)[:N]
    dei_col = dei2.reshape(M_PAD, 1)[:MY]

    xv = _matmul_a(X, W0, b0, dvi)
    ye0 = _smooth_p1(xv, vidx_r, eidx_r)
    xo0 = _smooth_p2(_combine(ye0, dei_col), vidx2_r, eidx2_r)
    zv = _matmul_b(xo0, W1, b1, dvi)
    ye1 = _smooth_p1(zv, vidx_r, eidx_r)
    xo1 = _smooth_p2(_combine(ye1, dei_col), vidx2_r, eidx2_r)
    out = _final_scale(xo1, dvi)
    return out[:, :N_CLS]


# speedup vs baseline: 1.1848x; 1.1848x over previous
import functools

import jax
import jax.numpy as jnp
from jax import lax
from jax.experimental import pallas as pl
from jax.experimental.pallas import tpu as pltpu
from jax.experimental.pallas import tpu_sc as plsc

N = 10000
M = 2500
E = 320000
D_IN = 128
D_HID = 128
N_CLS = 40

NC = 2
NS = 16
L = 16

N_PAD = 10240
M_PAD = 4096
NT = N_PAD // NS
MT = M_PAD // NS

EW = E // (NC * NS)
ROWS = EW // L

C = 100
EC = E // NS
NCH = EC // C
NB = 5

BN = 400

_sc_mesh = functools.partial(
    plsc.VectorSubcoreMesh, core_axis_name="c", subcore_axis_name="s",
    num_cores=NC, num_subcores=NS)


def _degrees_body(vidx_hbm, eidx_hbm, dv_out, de_out,
                  idxv, dvl, del_, dvsl, desl, rbv, rbe, rov, roe):
    c = lax.axis_index("c")
    s = lax.axis_index("s")
    w = c * NS + s
    zeros = jnp.zeros((L,), jnp.float32)
    ones = jnp.ones((L,), jnp.float32)

    def zero1d(ref, n):
        def zb(i, _):
            ref[pl.ds(i * L, L)] = zeros
            return 0
        lax.fori_loop(0, n // L, zb, 0)

    zero1d(dvl, N_PAD)
    zero1d(del_, M_PAD)

    pltpu.sync_copy(vidx_hbm.at[w], idxv)

    def hv(i, _):
        plsc.addupdate_scatter(dvl, [idxv[i]], ones)
        return 0
    lax.fori_loop(0, ROWS, hv, 0)

    pltpu.sync_copy(eidx_hbm.at[w], idxv)

    def he(i, _):
        plsc.addupdate_scatter(del_, [idxv[i]], ones)
        return 0
    lax.fori_loop(0, ROWS, he, 0)

    pltpu.sync_copy(dvl, dvsl.at[s])
    pltpu.sync_copy(del_, desl.at[s])
    plsc.subcore_barrier()

    pltpu.sync_copy(dvsl.at[:, pl.ds(s * NT, NT)], rbv)
    pltpu.sync_copy(desl.at[:, pl.ds(s * MT, MT)], rbe)

    def reduce_cols(rb, ro, nt):
        def red(j, _):
            def addk(k, acc):
                return acc + rb[k, pl.ds(j * L, L)]
            ro[pl.ds(j * L, L)] = lax.fori_loop(
                0, NS, addk, jnp.zeros((L,), jnp.float32))
            return 0
        lax.fori_loop(0, nt // L, red, 0)

    reduce_cols(rbv, rov, NT)
    reduce_cols(rbe, roe, MT)
    pltpu.sync_copy(rov, dv_out.at[c, pl.ds(s * NT, NT)])
    pltpu.sync_copy(roe, de_out.at[c, pl.ds(s * MT, MT)])


def _degrees(vidx, eidx):
    vidx_r = vidx.reshape(NC * NS, ROWS, L)
    eidx_r = eidx.reshape(NC * NS, ROWS, L)
    return pl.kernel(
        _degrees_body,
        out_type=[jax.ShapeDtypeStruct((NC, N_PAD), jnp.float32),
                  jax.ShapeDtypeStruct((NC, M_PAD), jnp.float32)],
        mesh=_sc_mesh(),
        compiler_params=pltpu.CompilerParams(needs_layout_passes=False),
        scratch_types=[
            pltpu.VMEM((ROWS, L), jnp.int32),
            pltpu.VMEM((N_PAD,), jnp.float32),
            pltpu.VMEM((M_PAD,), jnp.float32),
            pltpu.VMEM_SHARED((NS, N_PAD), jnp.float32),
            pltpu.VMEM_SHARED((NS, M_PAD), jnp.float32),
            pltpu.VMEM((NS, NT), jnp.float32),
            pltpu.VMEM((NS, MT), jnp.float32),
            pltpu.VMEM((NT,), jnp.float32),
            pltpu.VMEM((MT,), jnp.float32),
        ],
    )(vidx_r, eidx_r)


EWS = E // (NC * NS)
NCH = EWS // C
C2 = 40
NCH2 = EWS // C2
D = 128
MY = 2560
MYT = MY // NS
HALF = N_PAD // 2
HT = HALF // NS
TR = 128


def _zero_rowbuf0(rowbuf, nr, dw):
    zeros = jnp.zeros((L,), jnp.float32)

    def zb(i, _):
        rowbuf[0, i // (dw // L), pl.ds((i % (dw // L)) * L, L)] = zeros
        return 0
    lax.fori_loop(0, nr * dw // L, zb, 0)


def _pipeline(rowbuf, gsem, ssem, gsrc, gidx, sdst, sidx, nch):
    def issue_g(j, b):
        pltpu.async_copy(gsrc.at[gidx.at[j]], rowbuf.at[b], gsem.at[b])

    def wait_g(j, b):
        pltpu.make_async_copy(
            gsrc.at[gidx.at[j]], rowbuf.at[b], gsem.at[b]).wait()

    def issue_s(j, b):
        pltpu.async_copy(rowbuf.at[b], sdst.at[sidx.at[j]], ssem.at[b],
                         add=True)

    def wait_s(j, b):
        pltpu.make_async_copy(
            rowbuf.at[b], sdst.at[sidx.at[j]], ssem.at[b]).wait()

    for b in range(NB):
        issue_g(b, b)

    def body(i, _):
        j0 = i * NB
        for b in range(NB):
            wait_g(j0 + b, b)
            issue_s(j0 + b, b)
        for b in range(NB):
            wait_s(j0 + b, b)

            @pl.when(j0 + b + NB < nch)
            def _():
                issue_g(j0 + b + NB, b)
        return 0
    lax.fori_loop(0, nch // NB, body, 0)


def _p1_body(xv_hbm, vidx_hbm, eidx_hbm, ye_out,
             vidx_v, eidx_v, rowbuf, ye_sp, gsem, ssem, *, dw):
    c = lax.axis_index("c")
    s = lax.axis_index("s")
    w = c * NS + s

    pltpu.sync_copy(vidx_hbm.at[w], vidx_v)
    pltpu.sync_copy(eidx_hbm.at[w], eidx_v)

    _zero_rowbuf0(rowbuf, C, dw)
    base = s * MYT
    for t in range(MYT // C):
        pltpu.sync_copy(rowbuf.at[0], ye_sp.at[pl.ds(base + t * C, C)])
    if MYT % C:
        pltpu.sync_copy(rowbuf.at[0].at[pl.ds(0, MYT % C)],
                        ye_sp.at[pl.ds(base + (MYT // C) * C, MYT % C)])
    plsc.subcore_barrier()

    _pipeline(rowbuf, gsem, ssem, xv_hbm, vidx_v, ye_sp, eidx_v, NCH)
    plsc.subcore_barrier()

    pltpu.sync_copy(ye_sp.at[pl.ds(base, MYT)],
                    ye_out.at[c, pl.ds(base, MYT)])


def _p2_body(ye_hbm, vidx_hbm, eidx_hbm, xo_out,
             vring, ering, rowbuf, ye_sp, xo_sp,
             gsem, ssem, visem, eisem, *, dw):
    c = lax.axis_index("c")
    s = lax.axis_index("s")
    w = c * NS + s
    zeros = jnp.zeros((L,), jnp.float32)

    _zero_rowbuf0(rowbuf, C2, dw)
    nrow = N_PAD // NS
    for t in range(nrow // C2):
        pltpu.sync_copy(rowbuf.at[0], xo_sp.at[pl.ds(s * nrow + t * C2, C2)])

    pltpu.sync_copy(ye_hbm.at[pl.ds(s * MYT, MYT)],
                    ye_sp.at[pl.ds(s * MYT, MYT)])
    plsc.subcore_barrier()

    def issue_i(j, b):
        pltpu.async_copy(vidx_hbm.at[w, j], vring.at[b], visem.at[b])
        pltpu.async_copy(eidx_hbm.at[w, j], ering.at[b], eisem.at[b])

    def wait_i(j, b):
        pltpu.make_async_copy(
            vidx_hbm.at[w, j], vring.at[b], visem.at[b]).wait()
        pltpu.make_async_copy(
            eidx_hbm.at[w, j], ering.at[b], eisem.at[b]).wait()

    def issue_g(j, b):
        pltpu.async_copy(ye_sp.at[ering.at[b]], rowbuf.at[b], gsem.at[b])

    def wait_g(j, b):
        pltpu.make_async_copy(
            ye_sp.at[ering.at[b]], rowbuf.at[b], gsem.at[b]).wait()

    def issue_s(j, b):
        pltpu.async_copy(rowbuf.at[b], xo_sp.at[vring.at[b]], ssem.at[b],
                         add=True)

    def wait_s(j, b):
        pltpu.make_async_copy(
            rowbuf.at[b], xo_sp.at[vring.at[b]], ssem.at[b]).wait()

    for b in range(NB):
        issue_i(b, b)

    def body(i, _):
        j0 = i * NB
        for b in range(NB):
            wait_i(j0 + b, b)
            issue_g(j0 + b, b)
        for b in range(NB):
            wait_g(j0 + b, b)
            issue_s(j0 + b, b)
        for b in range(NB):
            wait_s(j0 + b, b)

            @pl.when(j0 + b + NB < NCH2)
            def _():
                issue_i(j0 + b + NB, b)
        return 0
    lax.fori_loop(0, NCH2 // NB, body, 0)
    plsc.subcore_barrier()

    pltpu.sync_copy(xo_sp.at[pl.ds(s * nrow, nrow)],
                    xo_out.at[c, pl.ds(s * nrow, nrow)])


def _smooth_p1(xv, vidx_r, eidx_r, dw=D, tiled=True):
    return pl.kernel(
        functools.partial(_p1_body, dw=dw),
        out_type=jax.ShapeDtypeStruct((NC, MY, dw), jnp.float32),
        mesh=_sc_mesh(),
        compiler_params=pltpu.CompilerParams(use_tc_tiling_on_sc=tiled),
        scratch_types=[
            pltpu.VMEM((NCH, C), jnp.int32),
            pltpu.VMEM((NCH, C), jnp.int32),
            pltpu.VMEM((NB, C, dw), jnp.float32),
            pltpu.VMEM_SHARED((MY, dw), jnp.float32),
            pltpu.SemaphoreType.DMA((NB,)),
            pltpu.SemaphoreType.DMA((NB,)),
        ],
    )(xv, vidx_r, eidx_r)


def _smooth_p2(ye_comb, vidx2_r, eidx2_r, dw=D, tiled=True):
    return pl.kernel(
        functools.partial(_p2_body, dw=dw),
        out_type=jax.ShapeDtypeStruct((NC, N_PAD, dw), jnp.float32),
        mesh=_sc_mesh(),
        compiler_params=pltpu.CompilerParams(use_tc_tiling_on_sc=tiled),
        scratch_types=[
            pltpu.VMEM((NB, C2), jnp.int32),
            pltpu.VMEM((NB, C2), jnp.int32),
            pltpu.VMEM((NB, C2, dw), jnp.float32),
            pltpu.VMEM_SHARED((MY, dw), jnp.float32),
            pltpu.VMEM_SHARED((N_PAD, dw), jnp.float32),
            pltpu.SemaphoreType.DMA((NB,)),
            pltpu.SemaphoreType.DMA((NB,)),
            pltpu.SemaphoreType.DMA((NB,)),
            pltpu.SemaphoreType.DMA((NB,)),
        ],
    )(ye_comb, vidx2_r, eidx2_r)


def _norm_body(dv_ref, de_ref, dvi_ref, dei_ref):
    dv = dv_ref[0:1, :] + dv_ref[1:2, :]
    de = de_ref[0:1, :] + de_ref[1:2, :]
    dvi_ref[...] = jnp.where(
        dv > 0, lax.rsqrt(jnp.maximum(dv, 1e-12)), 0.0)
    dei_ref[...] = jnp.where(de > 0, 1.0 / jnp.maximum(de, 1e-12), 0.0)


def _normalizers(dv_parts, de_parts):
    return pl.pallas_call(
        _norm_body,
        in_specs=[pl.BlockSpec((NC, N_PAD), lambda: (0, 0)),
                  pl.BlockSpec((NC, M_PAD), lambda: (0, 0))],
        out_specs=[pl.BlockSpec((1, N_PAD), lambda: (0, 0)),
                   pl.BlockSpec((1, M_PAD), lambda: (0, 0))],
        out_shape=[jax.ShapeDtypeStruct((1, N_PAD), jnp.float32),
                   jax.ShapeDtypeStruct((1, M_PAD), jnp.float32)],
    )(dv_parts, de_parts)


def _comb_body(p_ref, s_ref, o_ref):
    o_ref[...] = (p_ref[0] + p_ref[1]) * s_ref[...]


BM = 320


def _combine(ye_parts, dei_col, dw=D):
    return pl.pallas_call(
        _comb_body,
        grid=(MY // BM,),
        in_specs=[
            pl.BlockSpec((NC, BM, dw), lambda i: (0, i, 0)),
            pl.BlockSpec((BM, 1), lambda i: (i, 0)),
        ],
        out_specs=pl.BlockSpec((BM, dw), lambda i: (i, 0)),
        out_shape=jax.ShapeDtypeStruct((MY, dw), jnp.float32),
    )(ye_parts, dei_col)


def _mmA_body(x_ref, w_ref, b_ref, s_ref, o_ref):
    y = jnp.dot(x_ref[...], w_ref[...], preferred_element_type=jnp.float32)
    o_ref[...] = (y + b_ref[...]) * s_ref[...]


def _matmul_a(x, w, b, dvi):
    return pl.pallas_call(
        _mmA_body,
        grid=(N // BN,),
        in_specs=[
            pl.BlockSpec((BN, D_IN), lambda i: (i, 0)),
            pl.BlockSpec((D_IN, D_HID), lambda i: (0, 0)),
            pl.BlockSpec((1, D_HID), lambda i: (0, 0)),
            pl.BlockSpec((BN, 1), lambda i: (i, 0)),
        ],
        out_specs=pl.BlockSpec((BN, D_HID), lambda i: (i, 0)),
        out_shape=jax.ShapeDtypeStruct((N_PAD, D_HID), jnp.float32),
    )(x, w, b.reshape(1, D_HID), dvi)


def _mmB_body(x_ref, w_ref, b_ref, s_ref, o_ref):
    x = x_ref[0] + x_ref[1]
    h = jax.nn.relu(x * s_ref[...])
    y = jnp.dot(h, w_ref[...], preferred_element_type=jnp.float32)
    y = (y + b_ref[...]) * s_ref[...]
    o_ref[...] = jnp.concatenate(
        [y, jnp.zeros((BN, 48 - N_CLS), jnp.float32)], axis=1)


def _matmul_b(xo0, w, b, dvi):
    return pl.pallas_call(
        _mmB_body,
        grid=(N // BN,),
        in_specs=[
            pl.BlockSpec((NC, BN, D_HID), lambda i: (0, i, 0)),
            pl.BlockSpec((D_HID, N_CLS), lambda i: (0, 0)),
            pl.BlockSpec((1, N_CLS), lambda i: (0, 0)),
            pl.BlockSpec((BN, 1), lambda i: (i, 0)),
        ],
        out_specs=pl.BlockSpec((BN, 48), lambda i: (i, 0)),
        out_shape=jax.ShapeDtypeStruct((N_PAD, 48), jnp.float32),
    )(xo0, w, b.reshape(1, N_CLS), dvi)


def _final_body(x_ref, s_ref, o_ref):
    o_ref[...] = (x_ref[0] + x_ref[1]) * s_ref[...]


def _final_scale(xo1, dvi):
    return pl.pallas_call(
        _final_body,
        grid=(N // BN,),
        in_specs=[
            pl.BlockSpec((NC, BN, 48), lambda i: (0, i, 0)),
            pl.BlockSpec((BN, 1), lambda i: (i, 0)),
        ],
        out_specs=pl.BlockSpec((BN, 48), lambda i: (i, 0)),
        out_shape=jax.ShapeDtypeStruct((N, 48), jnp.float32),
    )(xo1, dvi)


def kernel(X, vertex_idx, hyperedge_idx, W0, b0, W1, b1):
    vidx = vertex_idx.astype(jnp.int32)
    eidx = hyperedge_idx.astype(jnp.int32)
    vidx_r = vidx.reshape(NC * NS, NCH, C)
    eidx_r = eidx.reshape(NC * NS, NCH, C)
    vidx2_r = vidx.reshape(NC * NS, NCH2, C2)
    eidx2_r = eidx.reshape(NC * NS, NCH2, C2)

    dv_parts, de_parts = _degrees(vidx, eidx)
    dvi2, dei2 = _normalizers(dv_parts, de_parts)
    dvi = dvi2.reshape(N_PAD, 1)[:N]
    dei_col = dei2.reshape(M_PAD, 1)[:MY]

    xv = _matmul_a(X, W0, b0, dvi)
    ye0 = _smooth_p1(xv, vidx_r, eidx_r)
    xo0 = _smooth_p2(_combine(ye0, dei_col), vidx2_r, eidx2_r)
    zv = _matmul_b(xo0, W1, b1, dvi)
    ye1 = _smooth_p1(zv, vidx_r, eidx_r, dw=48, tiled=False)
    xo1 = _smooth_p2(_combine(ye1, dei_col, dw=48), vidx2_r, eidx2_r,
                     dw=48, tiled=False)
    out = _final_scale(xo1, dvi)
    return out[:, :N_CLS]


# speedup vs baseline: 1.1854x; 1.0005x over previous
import functools

import jax
import jax.numpy as jnp
from jax import lax
from jax.experimental import pallas as pl
from jax.experimental.pallas import tpu as pltpu
from jax.experimental.pallas import tpu_sc as plsc

N = 10000
M = 2500
E = 320000
D_IN = 128
D_HID = 128
N_CLS = 40

NC = 2
NS = 16
L = 16

N_PAD = 10240
M_PAD = 4096
NT = N_PAD // NS
MT = M_PAD // NS

EW = E // (NC * NS)
ROWS = EW // L

C = 100
NB = 5

BN = 400

_sc_mesh = functools.partial(
    plsc.VectorSubcoreMesh, core_axis_name="c", subcore_axis_name="s",
    num_cores=NC, num_subcores=NS)


def _degrees_body(vidx_hbm, eidx_hbm, dv_out, de_out,
                  idxv, dvl, del_, dvsl, desl, rbv, rbe, rov, roe):
    c = lax.axis_index("c")
    s = lax.axis_index("s")
    w = c * NS + s
    zeros = jnp.zeros((L,), jnp.float32)
    ones = jnp.ones((L,), jnp.float32)

    def zero1d(ref, n):
        def zb(i, _):
            ref[pl.ds(i * L, L)] = zeros
            return 0
        lax.fori_loop(0, n // L, zb, 0)

    zero1d(dvl, N_PAD)
    zero1d(del_, M_PAD)

    pltpu.sync_copy(vidx_hbm.at[w], idxv)

    def hv(i, _):
        plsc.addupdate_scatter(dvl, [idxv[i]], ones)
        return 0
    lax.fori_loop(0, ROWS, hv, 0)

    pltpu.sync_copy(eidx_hbm.at[w], idxv)

    def he(i, _):
        plsc.addupdate_scatter(del_, [idxv[i]], ones)
        return 0
    lax.fori_loop(0, ROWS, he, 0)

    pltpu.sync_copy(dvl, dvsl.at[s])
    pltpu.sync_copy(del_, desl.at[s])
    plsc.subcore_barrier()

    pltpu.sync_copy(dvsl.at[:, pl.ds(s * NT, NT)], rbv)
    pltpu.sync_copy(desl.at[:, pl.ds(s * MT, MT)], rbe)

    def reduce_cols(rb, ro, nt):
        def red(j, _):
            def addk(k, acc):
                return acc + rb[k, pl.ds(j * L, L)]
            ro[pl.ds(j * L, L)] = lax.fori_loop(
                0, NS, addk, jnp.zeros((L,), jnp.float32))
            return 0
        lax.fori_loop(0, nt // L, red, 0)

    reduce_cols(rbv, rov, NT)
    reduce_cols(rbe, roe, MT)
    pltpu.sync_copy(rov, dv_out.at[c, pl.ds(s * NT, NT)])
    pltpu.sync_copy(roe, de_out.at[c, pl.ds(s * MT, MT)])


def _degrees(vidx, eidx):
    vidx_r = vidx.reshape(NC * NS, ROWS, L)
    eidx_r = eidx.reshape(NC * NS, ROWS, L)
    return pl.kernel(
        _degrees_body,
        out_type=[jax.ShapeDtypeStruct((NC, N_PAD), jnp.float32),
                  jax.ShapeDtypeStruct((NC, M_PAD), jnp.float32)],
        mesh=_sc_mesh(),
        compiler_params=pltpu.CompilerParams(needs_layout_passes=False),
        scratch_types=[
            pltpu.VMEM((ROWS, L), jnp.int32),
            pltpu.VMEM((N_PAD,), jnp.float32),
            pltpu.VMEM((M_PAD,), jnp.float32),
            pltpu.VMEM_SHARED((NS, N_PAD), jnp.float32),
            pltpu.VMEM_SHARED((NS, M_PAD), jnp.float32),
            pltpu.VMEM((NS, NT), jnp.float32),
            pltpu.VMEM((NS, MT), jnp.float32),
            pltpu.VMEM((NT,), jnp.float32),
            pltpu.VMEM((MT,), jnp.float32),
        ],
    )(vidx_r, eidx_r)


EWS = E // (NC * NS)
NCH = EWS // C
C2 = 40
NCH2 = EWS // C2
D = 128
MY = 2560
MYT = MY // NS
HALF = N_PAD // 2
HT = HALF // NS
TR = 128


def _zero_rowbuf0(rowbuf, nr, dw):
    zeros = jnp.zeros((L,), jnp.float32)

    def zb(i, _):
        rowbuf[0, i // (dw // L), pl.ds((i % (dw // L)) * L, L)] = zeros
        return 0
    lax.fori_loop(0, nr * dw // L, zb, 0)


def _pipeline(rowbuf, gsem, ssem, gsrc, gidx, sdst, sidx, nch):
    def issue_g(j, b):
        pltpu.async_copy(gsrc.at[gidx.at[j]], rowbuf.at[b], gsem.at[b])

    def wait_g(j, b):
        pltpu.make_async_copy(
            gsrc.at[gidx.at[j]], rowbuf.at[b], gsem.at[b]).wait()

    def issue_s(j, b):
        pltpu.async_copy(rowbuf.at[b], sdst.at[sidx.at[j]], ssem.at[b],
                         add=True)

    def wait_s(j, b):
        pltpu.make_async_copy(
            rowbuf.at[b], sdst.at[sidx.at[j]], ssem.at[b]).wait()

    for b in range(NB):
        issue_g(b, b)

    def body(i, _):
        j0 = i * NB
        for b in range(NB):
            wait_g(j0 + b, b)
            issue_s(j0 + b, b)
        for b in range(NB):
            wait_s(j0 + b, b)

            @pl.when(j0 + b + NB < nch)
            def _():
                issue_g(j0 + b + NB, b)
        return 0
    lax.fori_loop(0, nch // NB, body, 0)


def _p1_body(xv_hbm, vidx_hbm, eidx_hbm, ye_out,
             vidx_v, eidx_v, rowbuf, ye_sp, gsem, ssem, *, dw):
    c = lax.axis_index("c")
    s = lax.axis_index("s")
    w = c * NS + s

    pltpu.sync_copy(vidx_hbm.at[w], vidx_v)
    pltpu.sync_copy(eidx_hbm.at[w], eidx_v)

    _zero_rowbuf0(rowbuf, C, dw)
    base = s * MYT
    for t in range(MYT // C):
        pltpu.sync_copy(rowbuf.at[0], ye_sp.at[pl.ds(base + t * C, C)])
    if MYT % C:
        pltpu.sync_copy(rowbuf.at[0].at[pl.ds(0, MYT % C)],
                        ye_sp.at[pl.ds(base + (MYT // C) * C, MYT % C)])
    plsc.subcore_barrier()

    _pipeline(rowbuf, gsem, ssem, xv_hbm, vidx_v, ye_sp, eidx_v, NCH)
    plsc.subcore_barrier()

    pltpu.sync_copy(ye_sp.at[pl.ds(base, MYT)],
                    ye_out.at[c, pl.ds(base, MYT)])


def _p2_body(ye_hbm, vidx_hbm, eidx_hbm, xo_out,
             vring, ering, rowbuf, ye_sp, xo_sp,
             gsem, ssem, visem, eisem, *, dw):
    c = lax.axis_index("c")
    s = lax.axis_index("s")
    w = c * NS + s
    zeros = jnp.zeros((L,), jnp.float32)

    _zero_rowbuf0(rowbuf, C2, dw)
    nrow = N_PAD // NS
    for t in range(nrow // C2):
        pltpu.sync_copy(rowbuf.at[0], xo_sp.at[pl.ds(s * nrow + t * C2, C2)])

    pltpu.sync_copy(ye_hbm.at[pl.ds(s * MYT, MYT)],
                    ye_sp.at[pl.ds(s * MYT, MYT)])
    plsc.subcore_barrier()

    def issue_i(j, b):
        pltpu.async_copy(vidx_hbm.at[w, j], vring.at[b], visem.at[b])
        pltpu.async_copy(eidx_hbm.at[w, j], ering.at[b], eisem.at[b])

    def wait_i(j, b):
        pltpu.make_async_copy(
            vidx_hbm.at[w, j], vring.at[b], visem.at[b]).wait()
        pltpu.make_async_copy(
            eidx_hbm.at[w, j], ering.at[b], eisem.at[b]).wait()

    def issue_g(j, b):
        pltpu.async_copy(ye_sp.at[ering.at[b]], rowbuf.at[b], gsem.at[b])

    def wait_g(j, b):
        pltpu.make_async_copy(
            ye_sp.at[ering.at[b]], rowbuf.at[b], gsem.at[b]).wait()

    def issue_s(j, b):
        pltpu.async_copy(rowbuf.at[b], xo_sp.at[vring.at[b]], ssem.at[b],
                         add=True)

    def wait_s(j, b):
        pltpu.make_async_copy(
            rowbuf.at[b], xo_sp.at[vring.at[b]], ssem.at[b]).wait()

    for b in range(NB):
        issue_i(b, b)

    def body(i, _):
        j0 = i * NB
        for b in range(NB):
            wait_i(j0 + b, b)
            issue_g(j0 + b, b)
        for b in range(NB):
            wait_g(j0 + b, b)
            issue_s(j0 + b, b)
        for b in range(NB):
            wait_s(j0 + b, b)

            @pl.when(j0 + b + NB < NCH2)
            def _():
                issue_i(j0 + b + NB, b)
        return 0
    lax.fori_loop(0, NCH2 // NB, body, 0)
    plsc.subcore_barrier()

    pltpu.sync_copy(xo_sp.at[pl.ds(s * nrow, nrow)],
                    xo_out.at[c, pl.ds(s * nrow, nrow)])


def _smooth_p1(xv, vidx_r, eidx_r, dw=D, tiled=True):
    return pl.kernel(
        functools.partial(_p1_body, dw=dw),
        out_type=jax.ShapeDtypeStruct((NC, MY, dw), jnp.float32),
        mesh=_sc_mesh(),
        compiler_params=pltpu.CompilerParams(use_tc_tiling_on_sc=tiled),
        scratch_types=[
            pltpu.VMEM((NCH, C), jnp.int32),
            pltpu.VMEM((NCH, C), jnp.int32),
            pltpu.VMEM((NB, C, dw), jnp.float32),
            pltpu.VMEM_SHARED((MY, dw), jnp.float32),
            pltpu.SemaphoreType.DMA((NB,)),
            pltpu.SemaphoreType.DMA((NB,)),
        ],
    )(xv, vidx_r, eidx_r)


def _smooth_p2(ye_comb, vidx2_r, eidx2_r, dw=D, tiled=True):
    return pl.kernel(
        functools.partial(_p2_body, dw=dw),
        out_type=jax.ShapeDtypeStruct((NC, N_PAD, dw), jnp.float32),
        mesh=_sc_mesh(),
        compiler_params=pltpu.CompilerParams(use_tc_tiling_on_sc=tiled),
        scratch_types=[
            pltpu.VMEM((NB, C2), jnp.int32),
            pltpu.VMEM((NB, C2), jnp.int32),
            pltpu.VMEM((NB, C2, dw), jnp.float32),
            pltpu.VMEM_SHARED((MY, dw), jnp.float32),
            pltpu.VMEM_SHARED((N_PAD, dw), jnp.float32),
            pltpu.SemaphoreType.DMA((NB,)),
            pltpu.SemaphoreType.DMA((NB,)),
            pltpu.SemaphoreType.DMA((NB,)),
            pltpu.SemaphoreType.DMA((NB,)),
        ],
    )(ye_comb, vidx2_r, eidx2_r)


def _norm_body(dv_ref, de_ref, dvi_ref, dei_ref):
    dv = dv_ref[0:1, :] + dv_ref[1:2, :]
    de = de_ref[0:1, :] + de_ref[1:2, :]
    dvi_ref[...] = jnp.where(
        dv > 0, lax.rsqrt(jnp.maximum(dv, 1e-12)), 0.0)
    dei_ref[...] = jnp.where(de > 0, 1.0 / jnp.maximum(de, 1e-12), 0.0)


def _normalizers(dv_parts, de_parts):
    return pl.pallas_call(
        _norm_body,
        in_specs=[pl.BlockSpec((NC, N_PAD), lambda: (0, 0)),
                  pl.BlockSpec((NC, M_PAD), lambda: (0, 0))],
        out_specs=[pl.BlockSpec((1, N_PAD), lambda: (0, 0)),
                   pl.BlockSpec((1, M_PAD), lambda: (0, 0))],
        out_shape=[jax.ShapeDtypeStruct((1, N_PAD), jnp.float32),
                   jax.ShapeDtypeStruct((1, M_PAD), jnp.float32)],
    )(dv_parts, de_parts)


def _comb_body(p_ref, s_ref, o_ref):
    o_ref[...] = (p_ref[0] + p_ref[1]) * s_ref[...]


BM = 320


def _combine(ye_parts, dei_col, dw=D):
    return pl.pallas_call(
        _comb_body,
        grid=(MY // BM,),
        in_specs=[
            pl.BlockSpec((NC, BM, dw), lambda i: (0, i, 0)),
            pl.BlockSpec((BM, 1), lambda i: (i, 0)),
        ],
        out_specs=pl.BlockSpec((BM, dw), lambda i: (i, 0)),
        out_shape=jax.ShapeDtypeStruct((MY, dw), jnp.float32),
    )(ye_parts, dei_col)


def _mmA_body(x_ref, w_ref, b_ref, s_ref, o_ref):
    y = jnp.dot(x_ref[...], w_ref[...], preferred_element_type=jnp.float32)
    o_ref[...] = (y + b_ref[...]) * s_ref[...]


def _matmul_a(x, w, b, dvi):
    return pl.pallas_call(
        _mmA_body,
        grid=(N // BN,),
        in_specs=[
            pl.BlockSpec((BN, D_IN), lambda i: (i, 0)),
            pl.BlockSpec((D_IN, D_HID), lambda i: (0, 0)),
            pl.BlockSpec((1, D_HID), lambda i: (0, 0)),
            pl.BlockSpec((BN, 1), lambda i: (i, 0)),
        ],
        out_specs=pl.BlockSpec((BN, D_HID), lambda i: (i, 0)),
        out_shape=jax.ShapeDtypeStruct((N_PAD, D_HID), jnp.float32),
    )(x, w, b.reshape(1, D_HID), dvi)


def _mmB_body(x_ref, w_ref, b_ref, s_ref, o_ref):
    x = x_ref[0] + x_ref[1]
    h = jax.nn.relu(x * s_ref[...])
    y = jnp.dot(h, w_ref[...], preferred_element_type=jnp.float32)
    y = (y + b_ref[...]) * s_ref[...]
    o_ref[...] = jnp.concatenate(
        [y, jnp.zeros((BN, 48 - N_CLS), jnp.float32)], axis=1)


def _matmul_b(xo0, w, b, dvi):
    return pl.pallas_call(
        _mmB_body,
        grid=(N // BN,),
        in_specs=[
            pl.BlockSpec((NC, BN, D_HID), lambda i: (0, i, 0)),
            pl.BlockSpec((D_HID, N_CLS), lambda i: (0, 0)),
            pl.BlockSpec((1, N_CLS), lambda i: (0, 0)),
            pl.BlockSpec((BN, 1), lambda i: (i, 0)),
        ],
        out_specs=pl.BlockSpec((BN, 48), lambda i: (i, 0)),
        out_shape=jax.ShapeDtypeStruct((N_PAD, 48), jnp.float32),
    )(xo0, w, b.reshape(1, N_CLS), dvi)


def _final_body(x_ref, s_ref, o_ref):
    o_ref[...] = (x_ref[0] + x_ref[1]) * s_ref[...]


def _final_scale(xo1, dvi):
    return pl.pallas_call(
        _final_body,
        grid=(N // BN,),
        in_specs=[
            pl.BlockSpec((NC, BN, 48), lambda i: (0, i, 0)),
            pl.BlockSpec((BN, 1), lambda i: (i, 0)),
        ],
        out_specs=pl.BlockSpec((BN, 48), lambda i: (i, 0)),
        out_shape=jax.ShapeDtypeStruct((N, 48), jnp.float32),
    )(xo1, dvi)


def kernel(X, vertex_idx, hyperedge_idx, W0, b0, W1, b1):
    vidx = vertex_idx.astype(jnp.int32)
    eidx = hyperedge_idx.astype(jnp.int32)
    vidx_r = vidx.reshape(NC * NS, NCH, C)
    eidx_r = eidx.reshape(NC * NS, NCH, C)
    vidx2_r = vidx.reshape(NC * NS, NCH2, C2)
    eidx2_r = eidx.reshape(NC * NS, NCH2, C2)

    dv_parts, de_parts = _degrees(vidx, eidx)
    dvi2, dei2 = _normalizers(dv_parts, de_parts)
    dvi = dvi2.reshape(N_PAD, 1)[:N]
    dei_col = dei2.reshape(M_PAD, 1)[:MY]

    xv = _matmul_a(X, W0, b0, dvi)
    ye0 = _smooth_p1(xv, vidx_r, eidx_r)
    xo0 = _smooth_p2(_combine(ye0, dei_col), vidx2_r, eidx2_r)
    zv = _matmul_b(xo0, W1, b1, dvi)
    ye1 = _smooth_p1(zv, vidx_r, eidx_r, dw=48, tiled=False)
    xo1 = _smooth_p2(_combine(ye1, dei_col, dw=48), vidx2_r, eidx2_r,
                     dw=48, tiled=False)
    out = _final_scale(xo1, dvi)
    return out[:, :N_CLS]
